# baseline (device time: 18363 ns/iter reference)
import jax
import jax.numpy as jnp
from jax import lax
from jax.experimental import pallas as pl
from jax.experimental.pallas import tpu as pltpu


def kernel(ids, E):
    t = ids.shape[0]
    v, d = E.shape

    my_x = lax.axis_index("x")
    local = ids - my_x * v
    valid = (local >= 0) & (local < v)
    safe = jnp.where(valid, local, 0)
    partial = jnp.take(E, safe, axis=0) * valid[:, None].astype(E.dtype)

    def body(p_ref, out_ref, comm_ref, send_sem, recv_sem):
        mx = lax.axis_index("x")
        my = lax.axis_index("y")
        mz = lax.axis_index("z")
        nbr = (1 - mx, my, mz)

        barrier = pltpu.get_barrier_semaphore()
        pl.semaphore_signal(
            barrier, inc=1, device_id=nbr, device_id_type=pl.DeviceIdType.MESH
        )
        pl.semaphore_wait(barrier, 1)

        rdma = pltpu.make_async_remote_copy(
            src_ref=p_ref,
            dst_ref=comm_ref,
            send_sem=send_sem,
            recv_sem=recv_sem,
            device_id=nbr,
            device_id_type=pl.DeviceIdType.MESH,
        )
        rdma.start()
        rdma.wait()
        out_ref[...] = p_ref[...] + comm_ref[...]

    return pl.pallas_call(
        body,
        out_shape=jax.ShapeDtypeStruct((t, d), jnp.float32),
        in_specs=[pl.BlockSpec(memory_space=pltpu.VMEM)],
        out_specs=pl.BlockSpec(memory_space=pltpu.VMEM),
        scratch_shapes=[
            pltpu.VMEM((t, d), jnp.float32),
            pltpu.SemaphoreType.DMA,
            pltpu.SemaphoreType.DMA,
        ],
        compiler_params=pltpu.CompilerParams(collective_id=0),
    )(partial)


# device time: 16628 ns/iter; 1.1043x vs baseline; 1.1043x over previous
import jax
import jax.numpy as jnp
from jax import lax
from jax.experimental import pallas as pl
from jax.experimental.pallas import tpu as pltpu

K = 4


def kernel(ids, E):
    t = ids.shape[0]
    v, d = E.shape
    half = t // 2
    ch = half // K

    my_x = lax.axis_index("x")
    local = ids - my_x * v
    valid = (local >= 0) & (local < v)
    safe = jnp.where(valid, local, 0)
    partial = jnp.take(E, safe, axis=0) * valid[:, None].astype(E.dtype)

    def body(p_ref, out_ref, xbuf, ybuf, x_send, x_recv, y_send, y_recv):
        mx = lax.axis_index("x")
        my = lax.axis_index("y")
        mz = lax.axis_index("z")
        xnbr = (1 - mx, my, mz)
        ynbr = (mx, 1 - my, mz)

        barrier = pltpu.get_barrier_semaphore()
        pl.semaphore_signal(
            barrier, inc=1, device_id=xnbr, device_id_type=pl.DeviceIdType.MESH
        )
        pl.semaphore_signal(
            barrier, inc=1, device_id=ynbr, device_id_type=pl.DeviceIdType.MESH
        )
        pl.semaphore_wait(barrier, 2)

        base = my * half

        x_rdmas = []
        for k in range(K):
            rdma = pltpu.make_async_remote_copy(
                src_ref=p_ref.at[pl.ds(base + k * ch, ch), :],
                dst_ref=xbuf.at[pl.ds(k * ch, ch), :],
                send_sem=x_send.at[k],
                recv_sem=x_recv.at[k],
                device_id=xnbr,
                device_id_type=pl.DeviceIdType.MESH,
            )
            rdma.start()
            x_rdmas.append(rdma)

        y_rdmas = []
        for k in range(K):
            x_rdmas[k].wait_recv()
            rdma = pltpu.make_async_remote_copy(
                src_ref=xbuf.at[pl.ds(k * ch, ch), :],
                dst_ref=ybuf.at[pl.ds(k * ch, ch), :],
                send_sem=y_send.at[k],
                recv_sem=y_recv.at[k],
                device_id=ynbr,
                device_id_type=pl.DeviceIdType.MESH,
            )
            rdma.start()
            y_rdmas.append(rdma)

        out_ref[pl.ds(base, half), :] = p_ref[pl.ds(base, half), :] + xbuf[...]

        for k in range(K):
            y_rdmas[k].wait_recv()
        obase = (1 - my) * half
        out_ref[pl.ds(obase, half), :] = p_ref[pl.ds(obase, half), :] + ybuf[...]

        for k in range(K):
            x_rdmas[k].wait_send()
            y_rdmas[k].wait_send()

    return pl.pallas_call(
        body,
        out_shape=jax.ShapeDtypeStruct((t, d), jnp.float32),
        in_specs=[pl.BlockSpec(memory_space=pltpu.VMEM)],
        out_specs=pl.BlockSpec(memory_space=pltpu.VMEM),
        scratch_shapes=[
            pltpu.VMEM((half, d), jnp.float32),
            pltpu.VMEM((half, d), jnp.float32),
            pltpu.SemaphoreType.DMA((K,)),
            pltpu.SemaphoreType.DMA((K,)),
            pltpu.SemaphoreType.DMA((K,)),
            pltpu.SemaphoreType.DMA((K,)),
        ],
        compiler_params=pltpu.CompilerParams(collective_id=0),
    )(partial)
